# baseline (speedup 1.0000x reference)
import jax
import jax.numpy as jnp
from jax import lax
from jax.experimental import pallas as pl
from jax.experimental.pallas import tpu as pltpu
from jax.experimental.pallas import tpu_sc as plsc

N = 10000
H = 128
E = 320000
NG = 64
MAX_Z = 1000
NC = 2
NS = 16
NW = NC * NS

_MESH = plsc.VectorSubcoreMesh(core_axis_name="c", subcore_axis_name="s")

K = 128
CZ = 8
ZCH = 79
NZPAD = NW * CZ * K
ZOUT = ZCH * K


def _zgather_body(zm, ztab, out, idx_v, rows_v, sem):
    w = lax.axis_index("c") * NS + lax.axis_index("s")
    pltpu.sync_copy(zm.at[pl.ds(w * CZ, CZ)], idx_v)
    for j in range(CZ):
        orig = j * NW + w

        @pl.when(orig < ZCH)
        def _do():
            pltpu.async_copy(ztab.at[idx_v.at[j]], rows_v, sem).wait()
            pltpu.sync_copy(rows_v, out.at[pl.ds(orig * K, K)])


_zgather = pl.kernel(
    _zgather_body,
    out_type=jax.ShapeDtypeStruct((ZOUT, H), jnp.float32),
    mesh=_MESH,
    scratch_types=[
        pltpu.VMEM((CZ, K), jnp.int32),
        pltpu.VMEM((K, H), jnp.float32),
        pltpu.SemaphoreType.DMA,
    ],
)

CE = 80
EPAD = NW * CE * K
HALF = 5120
OPAD = 2 * HALF
NR = HALF + K
SLAB = NR // NS
OUTR = HALF // NS


def _agg_body(x, srcm, dstpm, zrows, out, idx_s, idx_p, idx_d,
              rows0, rows1, gsem, shared):
    c = lax.axis_index("c")
    s = lax.axis_index("s")
    shift = c * 16
    pltpu.sync_copy(zrows, shared.at[pl.ds(s * SLAB, SLAB)])

    def extract(j):
        for k in range(K // 16):
            v = idx_p[j, pl.ds(k * 16, 16)]
            idx_d[pl.ds(k * 16, 16)] = (v >> shift) & 0xFFFF

    def pair(i, carry):
        g = 2 * i
        pltpu.async_copy(x.at[idx_s.at[g + 1]], rows1, gsem)
        extract(g)
        pltpu.make_async_copy(x.at[idx_s.at[g]], rows0, gsem).wait()
        pltpu.sync_copy(rows0, shared.at[idx_d], add=True)

        @pl.when(g + 2 < CE)
        def _next():
            pltpu.async_copy(x.at[idx_s.at[g + 2]], rows0, gsem)

        extract(g + 1)
        pltpu.make_async_copy(x.at[idx_s.at[g + 1]], rows1, gsem).wait()
        pltpu.sync_copy(rows1, shared.at[idx_d], add=True)
        return carry

    for p in range(2):
        w = (c if p == 0 else 1 - c) * NS + s
        pltpu.sync_copy(srcm.at[pl.ds(w * CE, CE)], idx_s)
        pltpu.sync_copy(dstpm.at[pl.ds(w * CE, CE)], idx_p)
        if p == 0:
            plsc.subcore_barrier()
        pltpu.async_copy(x.at[idx_s.at[0]], rows0, gsem)
        lax.fori_loop(0, CE // 2, pair, 0)

    plsc.subcore_barrier()
    off = s * OUTR
    pltpu.sync_copy(shared.at[pl.ds(off, OUTR)],
                    out.at[pl.ds(c * HALF + off, OUTR)])


_aggregate = pl.kernel(
    _agg_body,
    out_type=jax.ShapeDtypeStruct((OPAD, H), jnp.float32),
    mesh=_MESH,
    scratch_types=[
        pltpu.VMEM((CE, K), jnp.int32),
        pltpu.VMEM((CE, K), jnp.int32),
        pltpu.VMEM((K,), jnp.int32),
        pltpu.VMEM((K, H), jnp.float32),
        pltpu.VMEM((K, H), jnp.float32),
        pltpu.SemaphoreType.DMA,
        pltpu.VMEM_SHARED((NR, H), jnp.float32),
    ],
)

BL = 1000
NBL = N // BL


def _layer_body(x_ref, a_ref, W1_ref, b1_ref, W2_ref, b2_ref,
                g_ref, be_ref, out_ref, h_scr, s_scr, ss_scr):
    ph = pl.program_id(0)
    b = pl.program_id(1)

    @pl.when(ph == 0)
    def _compute():
        @pl.when(b == 0)
        def _init():
            s_scr[...] = jnp.zeros_like(s_scr)
            ss_scr[...] = jnp.zeros_like(ss_scr)

        h = x_ref[...] + a_ref[...]
        h = jnp.dot(h.astype(jnp.bfloat16), W1_ref[...].astype(jnp.bfloat16),
                    preferred_element_type=jnp.float32) + b1_ref[...]
        h = jnp.maximum(h, 0.0)
        h = jnp.dot(h.astype(jnp.bfloat16), W2_ref[...].astype(jnp.bfloat16),
                    preferred_element_type=jnp.float32) + b2_ref[...]
        h = jnp.maximum(h, 0.0)
        h_scr[pl.ds(b * BL, BL), :] = h
        s_scr[...] = s_scr[...] + jnp.sum(h, axis=0, keepdims=True)

    @pl.when(ph == 1)
    def _var():
        m = s_scr[...] * (1.0 / N)
        d = h_scr[pl.ds(b * BL, BL), :] - m
        ss_scr[...] = ss_scr[...] + jnp.sum(d * d, axis=0, keepdims=True)

    @pl.when(ph == 2)
    def _normalize():
        m = s_scr[...] * (1.0 / N)
        v = ss_scr[...] * (1.0 / N)
        inv = lax.rsqrt(v + 1e-5)
        h = h_scr[pl.ds(b * BL, BL), :]
        out_ref[...] = (h - m) * inv * g_ref[...] + be_ref[...]


_row_spec = pl.BlockSpec((BL, H), lambda p, b: (jnp.where(p == 0, b, 0), 0))
_full_spec = pl.BlockSpec((H, H), lambda p, b: (0, 0))
_vec_spec = pl.BlockSpec((1, H), lambda p, b: (0, 0))

_layer = pl.pallas_call(
    _layer_body,
    grid=(3, NBL),
    in_specs=[_row_spec, _row_spec,
              _full_spec, _vec_spec, _full_spec, _vec_spec,
              _vec_spec, _vec_spec],
    out_specs=pl.BlockSpec((BL, H), lambda p, b: (jnp.where(p == 2, b, 0), 0)),
    out_shape=jax.ShapeDtypeStruct((N, H), jnp.float32),
    scratch_shapes=[
        pltpu.VMEM((N, H), jnp.float32),
        pltpu.VMEM((1, H), jnp.float32),
        pltpu.VMEM((1, H), jnp.float32),
    ],
    compiler_params=pltpu.CompilerParams(
        dimension_semantics=("arbitrary", "arbitrary")),
)


def _pool_body(x1_ref, x2_ref, x3_ref, bat_ref, lw1_ref, lb1_ref, lw2_ref,
               lb2_ref, out_ref, s_scr, c_scr):
    b = pl.program_id(0)

    @pl.when(b == 0)
    def _init():
        s_scr[...] = jnp.zeros_like(s_scr)
        c_scr[...] = jnp.zeros_like(c_scr)

    @pl.when(b < NBL)
    def _accum():
        bat = bat_ref[...]
        onehot = (bat == lax.broadcasted_iota(jnp.int32, (BL, NG), 1)
                  ).astype(jnp.float32)
        dn = (((0,), (0,)), ((), ()))
        for l, xr in enumerate((x1_ref, x2_ref, x3_ref)):
            s_scr[l] = s_scr[l] + lax.dot_general(
                onehot, xr[...], dn, preferred_element_type=jnp.float32, precision=lax.Precision.HIGHEST)
        c_scr[...] = c_scr[...] + lax.dot_general(
            onehot, jnp.ones((BL, H), jnp.float32), dn,
            preferred_element_type=jnp.float32, precision=lax.Precision.HIGHEST)

    @pl.when(b == NBL)
    def _head():
        recip = 1.0 / jnp.maximum(c_scr[...], 1.0)
        acc = jnp.zeros((NG, H), jnp.float32)
        for l in range(3):
            acc = acc + jnp.dot((s_scr[l] * recip).astype(jnp.bfloat16),
                                lw1_ref[l].astype(jnp.bfloat16),
                                preferred_element_type=jnp.float32)
        h = jnp.maximum(acc + lb1_ref[...], 0.0)
        out_ref[...] = (jnp.dot(h.astype(jnp.bfloat16),
                                lw2_ref[...].astype(jnp.bfloat16),
                                preferred_element_type=jnp.float32)
                        + lb2_ref[...])


def _clamped(b):
    return (jnp.minimum(b, NBL - 1), 0)


_pool = pl.pallas_call(
    _pool_body,
    grid=(NBL + 1,),
    in_specs=[
        pl.BlockSpec((BL, H), _clamped),
        pl.BlockSpec((BL, H), _clamped),
        pl.BlockSpec((BL, H), _clamped),
        pl.BlockSpec((BL, 1), _clamped),
        pl.BlockSpec((3, H, H), lambda b: (0, 0, 0)),
        pl.BlockSpec((1, H), lambda b: (0, 0)),
        pl.BlockSpec((H, 1), lambda b: (0, 0)),
        pl.BlockSpec((1, 1), lambda b: (0, 0)),
    ],
    out_specs=pl.BlockSpec((NG, 1), lambda b: (0, 0)),
    out_shape=jax.ShapeDtypeStruct((NG, 1), jnp.float32),
    scratch_shapes=[
        pltpu.VMEM((3, NG, H), jnp.float32),
        pltpu.VMEM((NG, H), jnp.float32),
    ],
    compiler_params=pltpu.CompilerParams(dimension_semantics=("arbitrary",)),
)


def kernel(z, edge_index, batch, ztab, W1_0, b1_0, W2_0, b2_0, g_0, be_0,
           W1_1, b1_1, W2_1, b2_1, g_1, be_1, W1_2, b1_2, W2_2, b2_2, g_2,
           be_2, lw1, lb1, lw2, lb2):
    z = z.astype(jnp.int32)
    src = edge_index[0].astype(jnp.int32)
    dst = edge_index[1].astype(jnp.int32)

    pad_z = jnp.arange(NZPAD - N, dtype=jnp.int32) % MAX_Z
    zm = (jnp.concatenate([z, pad_z]).reshape(CZ, NW, K)
          .transpose(1, 0, 2).reshape(NW * CZ, K))

    npad = EPAD - E
    pad_src = jnp.arange(npad, dtype=jnp.int32) % N
    srcm = jnp.concatenate([src, pad_src]).reshape(NW * CE, K)
    dstp = jnp.concatenate([dst, jnp.full((npad,), -1, jnp.int32)])
    dummy = HALF + (jnp.arange(EPAD, dtype=jnp.int32) % K)
    dst0 = jnp.where((dstp >= 0) & (dstp < HALF), dstp, dummy)
    dst1 = jnp.where(dstp >= HALF, dstp - HALF, dummy)
    dstpm = (dst0 | (dst1 << 16)).reshape(NW * CE, K)
    zrows = jnp.zeros((SLAB, H), jnp.float32)

    x = _zgather(zm, ztab)[:N]

    params = [(W1_0, b1_0, W2_0, b2_0, g_0, be_0),
              (W1_1, b1_1, W2_1, b2_1, g_1, be_1),
              (W1_2, b1_2, W2_2, b2_2, g_2, be_2)]
    xs = []
    for (W1, b1, W2, b2, g, be) in params:
        agg = _aggregate(x, srcm, dstpm, zrows)
        x = _layer(x, agg[:N], W1, b1.reshape(1, H), W2,
                   b2.reshape(1, H), g.reshape(1, H), be.reshape(1, H))
        xs.append(x)

    out = _pool(xs[0], xs[1], xs[2], batch.astype(jnp.int32).reshape(N, 1),
                lw1.reshape(3, H, H), lb1.reshape(1, H), lw2,
                lb2.reshape(1, 1))
    return out


# speedup vs baseline: 1.3874x; 1.3874x over previous
import jax
import jax.numpy as jnp
from jax import lax
from jax.experimental import pallas as pl
from jax.experimental.pallas import tpu as pltpu
from jax.experimental.pallas import tpu_sc as plsc

N = 10000
H = 128
E = 320000
NG = 64
MAX_Z = 1000
NC = 2
NS = 16
NW = NC * NS

_MESH = plsc.VectorSubcoreMesh(core_axis_name="c", subcore_axis_name="s")

K = 128
CZ = 8
ZCH = 79
NZPAD = NW * CZ * K
ZOUT = ZCH * K


def _zgather_body(zm, ztab, out, idx_v, rows_v, sem):
    w = lax.axis_index("c") * NS + lax.axis_index("s")
    pltpu.sync_copy(zm.at[pl.ds(w * CZ, CZ)], idx_v)
    for j in range(CZ):
        orig = j * NW + w

        @pl.when(orig < ZCH)
        def _do():
            pltpu.async_copy(ztab.at[idx_v.at[j]], rows_v, sem).wait()
            pltpu.sync_copy(rows_v, out.at[pl.ds(orig * K, K)])


_zgather = pl.kernel(
    _zgather_body,
    out_type=jax.ShapeDtypeStruct((ZOUT, H), jnp.float32),
    mesh=_MESH,
    scratch_types=[
        pltpu.VMEM((CZ, K), jnp.int32),
        pltpu.VMEM((K, H), jnp.float32),
        pltpu.SemaphoreType.DMA,
    ],
)

CE = 80
EPAD = NW * CE * K
HALF = 5120
OPAD = 2 * HALF
NR = HALF + K
SLAB = NR // NS
OUTR = HALF // NS


CBR = 84


def _agg_body(x, srcm, dstpm, zrows, out, idx_s, idx_p, idx_d,
              rows0, rows1, sbuf, dbuf, gsem, shared):
    c = lax.axis_index("c")
    s = lax.axis_index("s")
    shift = c * 16
    pltpu.sync_copy(zrows, shared.at[pl.ds(s * SLAB, SLAB)])

    def comp_row(r, cnt):
        for k in range(K // 16):
            srcv = idx_s[r, pl.ds(k * 16, 16)]
            pv = idx_p[r, pl.ds(k * 16, 16)]
            local = (pv >> shift) & 0xFFFF
            msk = local < HALF
            mi = msk.astype(jnp.int32)
            pos = cnt + plsc.cumsum(mi) - 1
            pr = pos >> 7
            pc = pos & 127
            plsc.store_scatter(sbuf, [pr, pc], srcv, mask=msk)
            plsc.store_scatter(dbuf, [pr, pc], local, mask=msk)
            cnt = cnt + jnp.sum(mi)
        return cnt

    def load_d(g):
        for k in range(K // 16):
            idx_d[pl.ds(k * 16, 16)] = dbuf[g, pl.ds(k * 16, 16)]

    def pair(i, carry):
        g = 2 * i
        nch = carry
        pltpu.async_copy(x.at[sbuf.at[g + 1]], rows1, gsem)
        load_d(g)
        pltpu.make_async_copy(x.at[sbuf.at[g]], rows0, gsem).wait()
        pltpu.sync_copy(rows0, shared.at[idx_d], add=True)

        @pl.when(g + 2 < nch)
        def _next():
            pltpu.async_copy(x.at[sbuf.at[g + 2]], rows0, gsem)

        load_d(g + 1)
        pltpu.make_async_copy(x.at[sbuf.at[g + 1]], rows1, gsem).wait()
        pltpu.sync_copy(rows1, shared.at[idx_d], add=True)
        return carry

    lanes = lax.iota(jnp.int32, 16)
    for p in range(2):
        w = (c if p == 0 else 1 - c) * NS + s
        pltpu.sync_copy(srcm.at[pl.ds(w * CE, CE)], idx_s)
        pltpu.sync_copy(dstpm.at[pl.ds(w * CE, CE)], idx_p)
        if p == 0:
            plsc.subcore_barrier()
        cnt = lax.fori_loop(0, CE, comp_row, jnp.int32(0))
        ones = lanes >= 0
        for t2 in range(256 // 16):
            pos = cnt + t2 * 16 + lanes
            pr = pos >> 7
            pc = pos & 127
            plsc.store_scatter(sbuf, [pr, pc], (lanes + t2 * 16) & 127,
                               mask=ones)
            plsc.store_scatter(dbuf, [pr, pc],
                               HALF + ((lanes + t2 * 16) & 127), mask=ones)
        nch = ((cnt + 255) // 256) * 2
        pltpu.async_copy(x.at[sbuf.at[0]], rows0, gsem)
        lax.fori_loop(0, nch // 2, pair, nch)

    plsc.subcore_barrier()
    off = s * OUTR
    pltpu.sync_copy(shared.at[pl.ds(off, OUTR)],
                    out.at[pl.ds(c * HALF + off, OUTR)])


_aggregate = pl.kernel(
    _agg_body,
    out_type=jax.ShapeDtypeStruct((OPAD, H), jnp.float32),
    mesh=_MESH,
    scratch_types=[
        pltpu.VMEM((CE, K), jnp.int32),
        pltpu.VMEM((CE, K), jnp.int32),
        pltpu.VMEM((K,), jnp.int32),
        pltpu.VMEM((K, H), jnp.float32),
        pltpu.VMEM((K, H), jnp.float32),
        pltpu.VMEM((CBR, K), jnp.int32),
        pltpu.VMEM((CBR, K), jnp.int32),
        pltpu.SemaphoreType.DMA,
        pltpu.VMEM_SHARED((NR, H), jnp.float32),
    ],
    compiler_params=pltpu.CompilerParams(needs_layout_passes=False),
)

BL = 1000
NBL = N // BL


def _layer_body(x_ref, a_ref, W1_ref, b1_ref, W2_ref, b2_ref,
                g_ref, be_ref, out_ref, h_scr, s_scr, ss_scr):
    ph = pl.program_id(0)
    b = pl.program_id(1)

    @pl.when(ph == 0)
    def _compute():
        @pl.when(b == 0)
        def _init():
            s_scr[...] = jnp.zeros_like(s_scr)
            ss_scr[...] = jnp.zeros_like(ss_scr)

        h = x_ref[...] + a_ref[...]
        h = jnp.dot(h.astype(jnp.bfloat16), W1_ref[...].astype(jnp.bfloat16),
                    preferred_element_type=jnp.float32) + b1_ref[...]
        h = jnp.maximum(h, 0.0)
        h = jnp.dot(h.astype(jnp.bfloat16), W2_ref[...].astype(jnp.bfloat16),
                    preferred_element_type=jnp.float32) + b2_ref[...]
        h = jnp.maximum(h, 0.0)
        h_scr[pl.ds(b * BL, BL), :] = h
        s_scr[...] = s_scr[...] + jnp.sum(h, axis=0, keepdims=True)

    @pl.when(ph == 1)
    def _var():
        m = s_scr[...] * (1.0 / N)
        d = h_scr[pl.ds(b * BL, BL), :] - m
        ss_scr[...] = ss_scr[...] + jnp.sum(d * d, axis=0, keepdims=True)

    @pl.when(ph == 2)
    def _normalize():
        m = s_scr[...] * (1.0 / N)
        v = ss_scr[...] * (1.0 / N)
        inv = lax.rsqrt(v + 1e-5)
        h = h_scr[pl.ds(b * BL, BL), :]
        out_ref[...] = (h - m) * inv * g_ref[...] + be_ref[...]


_row_spec = pl.BlockSpec((BL, H), lambda p, b: (jnp.where(p == 0, b, 0), 0))
_full_spec = pl.BlockSpec((H, H), lambda p, b: (0, 0))
_vec_spec = pl.BlockSpec((1, H), lambda p, b: (0, 0))

_layer = pl.pallas_call(
    _layer_body,
    grid=(3, NBL),
    in_specs=[_row_spec, _row_spec,
              _full_spec, _vec_spec, _full_spec, _vec_spec,
              _vec_spec, _vec_spec],
    out_specs=pl.BlockSpec((BL, H), lambda p, b: (jnp.where(p == 2, b, 0), 0)),
    out_shape=jax.ShapeDtypeStruct((N, H), jnp.float32),
    scratch_shapes=[
        pltpu.VMEM((N, H), jnp.float32),
        pltpu.VMEM((1, H), jnp.float32),
        pltpu.VMEM((1, H), jnp.float32),
    ],
    compiler_params=pltpu.CompilerParams(
        dimension_semantics=("arbitrary", "arbitrary")),
)


def _pool_body(x1_ref, x2_ref, x3_ref, bat_ref, lw1_ref, lb1_ref, lw2_ref,
               lb2_ref, out_ref, s_scr, c_scr):
    b = pl.program_id(0)

    @pl.when(b == 0)
    def _init():
        s_scr[...] = jnp.zeros_like(s_scr)
        c_scr[...] = jnp.zeros_like(c_scr)

    @pl.when(b < NBL)
    def _accum():
        bat = bat_ref[...]
        onehot = (bat == lax.broadcasted_iota(jnp.int32, (BL, NG), 1)
                  ).astype(jnp.float32)
        dn = (((0,), (0,)), ((), ()))
        for l, xr in enumerate((x1_ref, x2_ref, x3_ref)):
            s_scr[l] = s_scr[l] + lax.dot_general(
                onehot, xr[...], dn, preferred_element_type=jnp.float32, precision=lax.Precision.HIGHEST)
        c_scr[...] = c_scr[...] + lax.dot_general(
            onehot, jnp.ones((BL, H), jnp.float32), dn,
            preferred_element_type=jnp.float32, precision=lax.Precision.HIGHEST)

    @pl.when(b == NBL)
    def _head():
        recip = 1.0 / jnp.maximum(c_scr[...], 1.0)
        acc = jnp.zeros((NG, H), jnp.float32)
        for l in range(3):
            acc = acc + jnp.dot((s_scr[l] * recip).astype(jnp.bfloat16),
                                lw1_ref[l].astype(jnp.bfloat16),
                                preferred_element_type=jnp.float32)
        h = jnp.maximum(acc + lb1_ref[...], 0.0)
        out_ref[...] = (jnp.dot(h.astype(jnp.bfloat16),
                                lw2_ref[...].astype(jnp.bfloat16),
                                preferred_element_type=jnp.float32)
                        + lb2_ref[...])


def _clamped(b):
    return (jnp.minimum(b, NBL - 1), 0)


_pool = pl.pallas_call(
    _pool_body,
    grid=(NBL + 1,),
    in_specs=[
        pl.BlockSpec((BL, H), _clamped),
        pl.BlockSpec((BL, H), _clamped),
        pl.BlockSpec((BL, H), _clamped),
        pl.BlockSpec((BL, 1), _clamped),
        pl.BlockSpec((3, H, H), lambda b: (0, 0, 0)),
        pl.BlockSpec((1, H), lambda b: (0, 0)),
        pl.BlockSpec((H, 1), lambda b: (0, 0)),
        pl.BlockSpec((1, 1), lambda b: (0, 0)),
    ],
    out_specs=pl.BlockSpec((NG, 1), lambda b: (0, 0)),
    out_shape=jax.ShapeDtypeStruct((NG, 1), jnp.float32),
    scratch_shapes=[
        pltpu.VMEM((3, NG, H), jnp.float32),
        pltpu.VMEM((NG, H), jnp.float32),
    ],
    compiler_params=pltpu.CompilerParams(dimension_semantics=("arbitrary",)),
)


def kernel(z, edge_index, batch, ztab, W1_0, b1_0, W2_0, b2_0, g_0, be_0,
           W1_1, b1_1, W2_1, b2_1, g_1, be_1, W1_2, b1_2, W2_2, b2_2, g_2,
           be_2, lw1, lb1, lw2, lb2):
    z = z.astype(jnp.int32)
    src = edge_index[0].astype(jnp.int32)
    dst = edge_index[1].astype(jnp.int32)

    pad_z = jnp.arange(NZPAD - N, dtype=jnp.int32) % MAX_Z
    zm = (jnp.concatenate([z, pad_z]).reshape(CZ, NW, K)
          .transpose(1, 0, 2).reshape(NW * CZ, K))

    npad = EPAD - E
    pad_src = jnp.arange(npad, dtype=jnp.int32) % N
    srcm = jnp.concatenate([src, pad_src]).reshape(NW * CE, K)
    dstp = jnp.concatenate([dst, jnp.full((npad,), -1, jnp.int32)])
    dummy = HALF + (jnp.arange(EPAD, dtype=jnp.int32) % K)
    dst0 = jnp.where((dstp >= 0) & (dstp < HALF), dstp, dummy)
    dst1 = jnp.where(dstp >= HALF, dstp - HALF, dummy)
    dstpm = (dst0 | (dst1 << 16)).reshape(NW * CE, K)
    zrows = jnp.zeros((SLAB, H), jnp.float32)

    x = _zgather(zm, ztab)[:N]

    params = [(W1_0, b1_0, W2_0, b2_0, g_0, be_0),
              (W1_1, b1_1, W2_1, b2_1, g_1, be_1),
              (W1_2, b1_2, W2_2, b2_2, g_2, be_2)]
    xs = []
    for (W1, b1, W2, b2, g, be) in params:
        agg = _aggregate(x, srcm, dstpm, zrows)
        x = _layer(x, agg[:N], W1, b1.reshape(1, H), W2,
                   b2.reshape(1, H), g.reshape(1, H), be.reshape(1, H))
        xs.append(x)

    out = _pool(xs[0], xs[1], xs[2], batch.astype(jnp.int32).reshape(N, 1),
                lw1.reshape(3, H, H), lb1.reshape(1, H), lw2,
                lb2.reshape(1, 1))
    return out
